# baseline (speedup 1.0000x reference)
import functools

import jax
import jax.numpy as jnp
import numpy as np
from jax import lax
from jax.experimental import pallas as pl
from jax.experimental.pallas import tpu as pltpu
from jax.experimental.pallas import tpu_sc as plsc

_N, _H, _W = 8, 32, 32
_L = _H * _W
_NL = _N * _L
_CH, _G = 192, 12
_GC = _CH // _G
_P = 9
_OMD = int(np.ceil(_G * _P * 3 / 8) * 8)
_ROWS_A = _CH + _G * 32
_CB = _NL // 4

_perm = np.zeros((_G * 32,), np.int32)
_keep = np.zeros((_G * 32, 1), np.float32)
_bshift = np.zeros((_G * 32,), np.float32)
for _g in range(_G):
    for _r in range(27):
        if _r < 9:
            _m = 2 * _r
            _bshift[_g * 32 + _r] = _r % 3 - 1
        elif _r < 18:
            _m = 2 * (_r - 9) + 1
            _bshift[_g * 32 + _r] = (_r - 9) // 3 - 1
        else:
            _m = _r
        _perm[_g * 32 + _r] = _g * 27 + _m
        _keep[_g * 32 + _r, 0] = 1.0


def _proj_in_body(w_ref, x_ref, b_ref, o_ref):
    o_ref[...] = (
        lax.dot_general(
            w_ref[...],
            x_ref[...],
            (((1,), (1,)), ((), ())),
            preferred_element_type=jnp.float32,
        )
        + b_ref[...]
    )


def _proj_in(w, x2, b):
    return pl.pallas_call(
        _proj_in_body,
        grid=(4,),
        in_specs=[
            pl.BlockSpec((_ROWS_A, _CH), lambda i: (0, 0)),
            pl.BlockSpec((_CB, _CH), lambda i: (i, 0)),
            pl.BlockSpec((_ROWS_A, 1), lambda i: (0, 0)),
        ],
        out_specs=pl.BlockSpec((_ROWS_A, _CB), lambda i: (0, i)),
        out_shape=jax.ShapeDtypeStruct((_ROWS_A, _NL), jnp.float32),
    )(w, x2, b)


def _proj_out_body(s_ref, w_ref, b_ref, o_ref):
    o_ref[...] = (
        lax.dot_general(
            s_ref[...],
            w_ref[...],
            (((0,), (1,)), ((), ())),
            preferred_element_type=jnp.float32,
        )
        + b_ref[...]
    )


def _proj_out(s, w, b):
    return pl.pallas_call(
        _proj_out_body,
        grid=(4,),
        in_specs=[
            pl.BlockSpec((_CH, _CB), lambda i: (0, i)),
            pl.BlockSpec((_CH, _CH), lambda i: (0, 0)),
            pl.BlockSpec((1, _CH), lambda i: (0, 0)),
        ],
        out_specs=pl.BlockSpec((_CB, _CH), lambda i: (i, 0)),
        out_shape=jax.ShapeDtypeStruct((_NL, _CH), jnp.float32),
    )(s, w, b)


_mesh = plsc.VectorSubcoreMesh(core_axis_name="c", subcore_axis_name="s")


@functools.partial(
    pl.kernel,
    mesh=_mesh,
    out_type=jax.ShapeDtypeStruct((_CH, _NL), jnp.float32),
    scratch_types=[
        pltpu.VMEM((_GC, _L), jnp.float32),
        pltpu.VMEM((32, _L), jnp.float32),
        pltpu.VMEM((_GC, _L), jnp.float32),
    ],
    compiler_params=pltpu.CompilerParams(
        use_tc_tiling_on_sc=False, needs_layout_passes=False
    ),
)
def _sc_sample(a_hbm, out_hbm, xvt, comp, outv):
    wid = lax.axis_index("s") * 2 + lax.axis_index("c")
    n = wid >> 2
    j = wid & 3

    def tbody(t, carry):
        g = j * 3 + t
        pltpu.sync_copy(a_hbm.at[pl.ds(g * _GC, _GC), pl.ds(n * _L, _L)], xvt)
        pltpu.sync_copy(
            a_hbm.at[pl.ds(_CH + g * 32, 32), pl.ds(n * _L, _L)], comp
        )

        def bbody(b, c2):
            l0 = b * 16
            li = lax.broadcasted_iota(jnp.int32, (16,), 0) + l0
            pix_y = (li >> 5).astype(jnp.float32)
            pix_x = (li & 31).astype(jnp.float32)
            acc = [jnp.zeros((16,), jnp.float32) for _ in range(_GC)]
            for p in range(_P):
                locx = pix_x + comp[p, pl.ds(l0, 16)]
                locy = pix_y + comp[9 + p, pl.ds(l0, 16)]
                msk = comp[18 + p, pl.ds(l0, 16)]
                xi = locx.astype(jnp.int32)
                yi = locy.astype(jnp.int32)
                xf = xi - jnp.where(locx < xi.astype(jnp.float32), 1, 0)
                yf = yi - jnp.where(locy < yi.astype(jnp.float32), 1, 0)
                lx = locx - xf.astype(jnp.float32)
                ly = locy - yf.astype(jnp.float32)
                hx = 1.0 - lx
                hym = (1.0 - ly) * msk
                lym = ly * msk
                x1 = xf + 1
                y1 = yf + 1
                vx0 = (xf >= 0) & (xf < _W)
                vx1 = (x1 >= 0) & (x1 < _W)
                vy0 = (yf >= 0) & (yf < _H)
                vy1 = (y1 >= 0) & (y1 < _H)
                xc0 = jnp.clip(xf, 0, _W - 1)
                xc1 = jnp.clip(x1, 0, _W - 1)
                ly0 = jnp.clip(yf, 0, _H - 1) * _W
                ly1 = jnp.clip(y1, 0, _H - 1) * _W
                for liny, vy, xc, vx, bw in (
                    (ly0, vy0, xc0, vx0, hym * hx),
                    (ly0, vy0, xc1, vx1, hym * lx),
                    (ly1, vy1, xc0, vx0, lym * hx),
                    (ly1, vy1, xc1, vx1, lym * lx),
                ):
                    lin = liny + xc
                    wv = jnp.where(vy & vx, bw, 0.0)
                    for ch in range(_GC):
                        cvec = jnp.full((16,), ch, jnp.int32)
                        val = plsc.load_gather(xvt, [cvec, lin])
                        acc[ch] = acc[ch] + wv * val
            for ch in range(_GC):
                outv[ch, pl.ds(l0, 16)] = acc[ch]
            return c2

        lax.fori_loop(0, _L // 16, bbody, 0)
        pltpu.sync_copy(
            outv, out_hbm.at[pl.ds(g * _GC, _GC), pl.ds(n * _L, _L)]
        )
        return carry

    lax.fori_loop(0, 3, tbody, 0)


def kernel(input, value_w, value_b, offset_mask_w, offset_mask_b, output_w, output_b):
    x2 = input.reshape(_NL, _CH)
    w2 = offset_mask_w[_perm] * _keep
    b2 = offset_mask_b[_perm] * _keep[:, 0] + _bshift
    w_all = jnp.concatenate([value_w, w2], axis=0)
    b_all = jnp.concatenate([value_b, b2], axis=0)[:, None]
    a = _proj_in(w_all, x2, b_all)
    s = _sc_sample(a)
    c = _proj_out(s, output_w, output_b[None, :])
    return c.reshape(_N, _L, _CH)
